# Initial kernel scaffold; baseline (speedup 1.0000x reference)
import jax
import jax.numpy as jnp
from jax.experimental import pallas as pl


def kernel(feats, edge_dict, W_fc, conv_w, conv_b):
    raise NotImplementedError("write your pallas kernel here")


# speedup vs baseline: 28.9998x; 28.9998x over previous
import functools

import jax
import jax.numpy as jnp
from jax import lax
from jax.experimental import pallas as pl
from jax.experimental.pallas import tpu as pltpu
from jax.experimental.pallas import tpu_sc as plsc

N = 10000
D = 128
KN = 3
NPAD = 10240
RB = 256
NRB = NPAD // RB
CT = 512
NCT = NPAD // CT
NEG = float("-inf")
BIGI = 2**30


def _topk_body(feats_full_ref, rows_ref, idx_ref, xn_ref, cv_ref, ci_ref):
    r = pl.program_id(0)

    @pl.when(r == 0)
    def _():
        for c in range(NRB):
            x = feats_full_ref[pl.ds(c * RB, RB), :]
            n2 = jnp.sum(x * x, axis=1, keepdims=True)
            xn_ref[pl.ds(c * RB, RB), :] = x / jnp.maximum(jnp.sqrt(n2), 1e-12)

    x = rows_ref[...]
    n2 = jnp.sum(x * x, axis=1, keepdims=True)
    rows = x / jnp.maximum(jnp.sqrt(n2), 1e-12)

    cv_ref[...] = jnp.full((RB, 16), NEG, dtype=jnp.float32)
    ci_ref[...] = jnp.zeros((RB, 16), dtype=jnp.int32)

    for j in range(NCT):
        cols = xn_ref[pl.ds(j * CT, CT), :]
        s = lax.dot_general(rows, cols, (((1,), (1,)), ((), ())),
                            preferred_element_type=jnp.float32)
        colid = lax.broadcasted_iota(jnp.int32, (RB, CT), 1) + j * CT
        s = jnp.where(colid < N, s, NEG)
        v = s
        for t in range(3):
            m = jnp.max(v, axis=1, keepdims=True)
            am = jnp.min(jnp.where(v == m, colid, BIGI), axis=1, keepdims=True)
            cv_ref[:, 3 + t:4 + t] = m
            ci_ref[:, 3 + t:4 + t] = am
            v = jnp.where(colid == am, NEG, v)
        cv = cv_ref[...]
        ci = ci_ref[...]
        nv, ni = [], []
        for t in range(3):
            m = jnp.max(cv, axis=1, keepdims=True)
            am = jnp.min(jnp.where(cv == m, ci, BIGI), axis=1, keepdims=True)
            nv.append(m)
            ni.append(am)
            cv = jnp.where(ci == am, NEG, cv)
        cv_ref[:, 0:3] = jnp.concatenate(nv, axis=1)
        ci_ref[:, 0:3] = jnp.concatenate(ni, axis=1)

    idx_ref[...] = ci_ref[:, 0:8]


def _run_topk(feats_pad):
    return pl.pallas_call(
        _topk_body,
        grid=(NRB,),
        in_specs=[
            pl.BlockSpec((NPAD, D), lambda r: (0, 0)),
            pl.BlockSpec((RB, D), lambda r: (r, 0)),
        ],
        out_specs=pl.BlockSpec((RB, 8), lambda r: (r, 0)),
        out_shape=jax.ShapeDtypeStruct((NPAD, 8), jnp.int32),
        scratch_shapes=[
            pltpu.VMEM((NPAD, D), jnp.float32),
            pltpu.VMEM((RB, 16), jnp.float32),
            pltpu.VMEM((RB, 16), jnp.int32),
        ],
    )(feats_pad, feats_pad)


def _make_gather():
    info = plsc.get_sparse_core_info()
    nw = info.num_cores * info.num_subcores
    bpw = NPAD // nw
    mesh = plsc.VectorSubcoreMesh(core_axis_name="c", subcore_axis_name="s")

    @functools.partial(
        pl.kernel,
        mesh=mesh,
        out_type=jax.ShapeDtypeStruct((KN, NPAD, D), jnp.float32),
        scratch_types=[
            pltpu.VMEM((bpw,), jnp.int32),
            pltpu.VMEM((bpw, D), jnp.float32),
            pltpu.SemaphoreType.DMA,
        ],
    )
    def gather_k(table_hbm, i0_hbm, i1_hbm, i2_hbm, out_hbm, idx_v, rows_v,
                 sem):
        wid = lax.axis_index("s") * info.num_cores + lax.axis_index("c")
        base = wid * bpw
        for k, idx_hbm in enumerate((i0_hbm, i1_hbm, i2_hbm)):
            pltpu.sync_copy(idx_hbm.at[pl.ds(base, bpw)], idx_v)
            pltpu.async_copy(table_hbm.at[idx_v], rows_v, sem).wait()
            pltpu.sync_copy(rows_v, out_hbm.at[k, pl.ds(base, bpw)])

    return gather_k


def _fc_body(g_ref, w_ref, p_ref, out_ref):
    g0 = g_ref[0]
    g1 = g_ref[1]
    g2 = g_ref[2]
    pooled = (g0 * p_ref[0, 0] + g1 * p_ref[0, 1] + g2 * p_ref[0, 2]
              + p_ref[0, 3])
    acc = lax.dot_general(pooled, w_ref[...], (((1,), (1,)), ((), ())),
                          preferred_element_type=jnp.float32)
    out_ref[...] = jnp.maximum(acc, 0.0)


def _run_fc(g, w_fc, params):
    return pl.pallas_call(
        _fc_body,
        grid=(NRB,),
        in_specs=[
            pl.BlockSpec((KN, RB, D), lambda r: (0, r, 0)),
            pl.BlockSpec((D, D), lambda r: (0, 0)),
            pl.BlockSpec(memory_space=pltpu.SMEM),
        ],
        out_specs=pl.BlockSpec((RB, D), lambda r: (r, 0)),
        out_shape=jax.ShapeDtypeStruct((NPAD, D), jnp.float32),
    )(g, w_fc, params)


def kernel(feats, edge_dict, W_fc, conv_w, conv_b):
    del edge_dict
    f2 = feats[0]
    fpad = jnp.pad(f2, ((0, NPAD - N), (0, 0)))

    idx8 = _run_topk(fpad)
    i0 = idx8[:, 0]
    i1 = idx8[:, 1]
    i2 = idx8[:, 2]

    g = _make_gather()(f2, i0, i1, i2)

    params = jnp.zeros((1, 8), jnp.float32)
    params = params.at[0, :KN].set(conv_w).at[0, KN].set(conv_b[0])
    out = _run_fc(g, W_fc, params)
    return out[:N][None]


# speedup vs baseline: 49.2645x; 1.6988x over previous
import functools

import jax
import jax.numpy as jnp
from jax import lax
from jax.experimental import pallas as pl
from jax.experimental.pallas import tpu as pltpu
from jax.experimental.pallas import tpu_sc as plsc

N = 10000
D = 128
KN = 3
NPAD = 10240
RB = 256
NRB = NPAD // RB
CT = 1024
NCT = NPAD // CT
NEG = float("-inf")
BIGI = 2**30


def _top3_of(v, ids, pad_id):
    vals, outs = [], []
    for _ in range(3):
        m = jnp.max(v, axis=1, keepdims=True)
        am = jnp.min(jnp.where(v == m, ids, pad_id), axis=1, keepdims=True)
        vals.append(m)
        outs.append(am)
        v = jnp.where(ids == am, NEG, v)
    return vals, outs


def _topk_body(feats_full_ref, rows_ref, idx_ref, xn_ref):
    r = pl.program_id(0)

    @pl.when(r == 0)
    def _():
        for c in range(NRB):
            x = feats_full_ref[pl.ds(c * RB, RB), :]
            n2 = jnp.sum(x * x, axis=1, keepdims=True)
            xn_ref[pl.ds(c * RB, RB), :] = x / jnp.maximum(jnp.sqrt(n2), 1e-12)

    x = rows_ref[...]
    n2 = jnp.sum(x * x, axis=1, keepdims=True)
    rows = x / jnp.maximum(jnp.sqrt(n2), 1e-12)

    lid = lax.broadcasted_iota(jnp.int32, (RB, CT), 1)
    run_v = [jnp.full((RB, 1), NEG, dtype=jnp.float32)] * 3
    run_i = [jnp.zeros((RB, 1), dtype=jnp.int32)] * 3

    for j in range(NCT):
        cols = xn_ref[pl.ds(j * CT, CT), :]
        s = lax.dot_general(rows, cols, (((1,), (1,)), ((), ())),
                            preferred_element_type=jnp.float32)
        if (j + 1) * CT > N:
            s = jnp.where(lid < N - j * CT, s, NEG)
        tv, ti = _top3_of(s, lid, BIGI)
        ti = [t + j * CT for t in ti]
        pad = jnp.full((RB, 1), NEG, dtype=jnp.float32)
        padi = jnp.full((RB, 1), BIGI, dtype=jnp.int32)
        cv = jnp.concatenate(run_v + tv + [pad, pad], axis=1)
        ci = jnp.concatenate(run_i + ti + [padi, padi], axis=1)
        run_v, run_i = _top3_of(cv, ci, BIGI)

    padi = jnp.full((RB, 1), 0, dtype=jnp.int32)
    idx_ref[...] = jnp.concatenate(run_i + [padi] * 5, axis=1)


def _run_topk(feats_pad):
    return pl.pallas_call(
        _topk_body,
        grid=(NRB,),
        in_specs=[
            pl.BlockSpec((NPAD, D), lambda r: (0, 0)),
            pl.BlockSpec((RB, D), lambda r: (r, 0)),
        ],
        out_specs=pl.BlockSpec((RB, 8), lambda r: (r, 0)),
        out_shape=jax.ShapeDtypeStruct((NPAD, 8), jnp.int32),
        scratch_shapes=[
            pltpu.VMEM((NPAD, D), jnp.float32),
        ],
    )(feats_pad, feats_pad)


def _make_gather():
    info = plsc.get_sparse_core_info()
    nw = info.num_cores * info.num_subcores
    bpw = NPAD // nw
    mesh = plsc.VectorSubcoreMesh(core_axis_name="c", subcore_axis_name="s")

    @functools.partial(
        pl.kernel,
        mesh=mesh,
        out_type=jax.ShapeDtypeStruct((KN, NPAD, D), jnp.float32),
        scratch_types=[
            pltpu.VMEM((bpw,), jnp.int32),
            pltpu.VMEM((bpw, D), jnp.float32),
            pltpu.SemaphoreType.DMA,
        ],
    )
    def gather_k(table_hbm, i0_hbm, i1_hbm, i2_hbm, out_hbm, idx_v, rows_v,
                 sem):
        wid = lax.axis_index("s") * info.num_cores + lax.axis_index("c")
        base = wid * bpw
        for k, idx_hbm in enumerate((i0_hbm, i1_hbm, i2_hbm)):
            pltpu.sync_copy(idx_hbm.at[pl.ds(base, bpw)], idx_v)
            pltpu.async_copy(table_hbm.at[idx_v], rows_v, sem).wait()
            pltpu.sync_copy(rows_v, out_hbm.at[k, pl.ds(base, bpw)])

    return gather_k


def _fc_body(g_ref, w_ref, p_ref, out_ref):
    g0 = g_ref[0]
    g1 = g_ref[1]
    g2 = g_ref[2]
    pooled = (g0 * p_ref[0, 0] + g1 * p_ref[0, 1] + g2 * p_ref[0, 2]
              + p_ref[0, 3])
    acc = lax.dot_general(pooled, w_ref[...], (((1,), (1,)), ((), ())),
                          preferred_element_type=jnp.float32)
    out_ref[...] = jnp.maximum(acc, 0.0)


def _run_fc(g, w_fc, params):
    return pl.pallas_call(
        _fc_body,
        grid=(NRB,),
        in_specs=[
            pl.BlockSpec((KN, RB, D), lambda r: (0, r, 0)),
            pl.BlockSpec((D, D), lambda r: (0, 0)),
            pl.BlockSpec(memory_space=pltpu.SMEM),
        ],
        out_specs=pl.BlockSpec((RB, D), lambda r: (r, 0)),
        out_shape=jax.ShapeDtypeStruct((NPAD, D), jnp.float32),
    )(g, w_fc, params)


def kernel(feats, edge_dict, W_fc, conv_w, conv_b):
    del edge_dict
    f2 = feats[0]
    fpad = jnp.pad(f2, ((0, NPAD - N), (0, 0)))

    idx8 = _run_topk(fpad)
    i0 = idx8[:, 0]
    i1 = idx8[:, 1]
    i2 = idx8[:, 2]

    g = _make_gather()(f2, i0, i1, i2)

    params = jnp.zeros((1, 8), jnp.float32)
    params = params.at[0, :KN].set(conv_w).at[0, KN].set(conv_b[0])
    out = _run_fc(g, W_fc, params)
    return out[:N][None]


# speedup vs baseline: 60.0841x; 1.2196x over previous
import functools

import jax
import jax.numpy as jnp
from jax import lax
from jax.experimental import pallas as pl
from jax.experimental.pallas import tpu as pltpu
from jax.experimental.pallas import tpu_sc as plsc

N = 10000
D = 128
KN = 3
NPAD = 10240
RB = 512
NRB = NPAD // RB
CT = 1024
NCT = NPAD // CT
NEG = float("-inf")
BIGI = 2**30


def _top3_of(v, ids, pad_id):
    vals, outs = [], []
    for _ in range(3):
        m = jnp.max(v, axis=1, keepdims=True)
        am = jnp.min(jnp.where(v == m, ids, pad_id), axis=1, keepdims=True)
        vals.append(m)
        outs.append(am)
        v = jnp.where(ids == am, NEG, v)
    return vals, outs


def _topk_body(feats_full_ref, rows_ref, idx_ref, xn_ref):
    r = pl.program_id(0)

    @pl.when(r == 0)
    def _():
        for c in range(NRB):
            x = feats_full_ref[pl.ds(c * RB, RB), :]
            n2 = jnp.sum(x * x, axis=1, keepdims=True)
            xn_ref[pl.ds(c * RB, RB), :] = x / jnp.maximum(jnp.sqrt(n2), 1e-12)

    x = rows_ref[...]
    n2 = jnp.sum(x * x, axis=1, keepdims=True)
    rows = x / jnp.maximum(jnp.sqrt(n2), 1e-12)

    lid = lax.broadcasted_iota(jnp.int32, (RB, CT), 1)
    run_v, run_i = None, None

    for j in range(NCT):
        cols = xn_ref[pl.ds(j * CT, CT), :]
        s = lax.dot_general(rows, cols, (((1,), (1,)), ((), ())),
                            preferred_element_type=jnp.float32)
        if (j + 1) * CT > N:
            s = jnp.where(lid < N - j * CT, s, NEG)
        tv, ti = _top3_of(s, lid, BIGI)
        ti = [t + j * CT for t in ti]
        if j == 0:
            run_v, run_i = tv, ti
            continue
        pad = jnp.full((RB, 1), NEG, dtype=jnp.float32)
        padi = jnp.full((RB, 1), BIGI, dtype=jnp.int32)
        cv = jnp.concatenate(run_v + tv + [pad, pad], axis=1)
        ci = jnp.concatenate(run_i + ti + [padi, padi], axis=1)
        run_v, run_i = _top3_of(cv, ci, BIGI)

    padi = jnp.full((RB, 1), 0, dtype=jnp.int32)
    idx_ref[...] = jnp.concatenate(run_i + [padi] * 5, axis=1)


def _run_topk(feats_pad):
    return pl.pallas_call(
        _topk_body,
        grid=(NRB,),
        in_specs=[
            pl.BlockSpec((NPAD, D), lambda r: (0, 0)),
            pl.BlockSpec((RB, D), lambda r: (r, 0)),
        ],
        out_specs=pl.BlockSpec((RB, 8), lambda r: (r, 0)),
        out_shape=jax.ShapeDtypeStruct((NPAD, 8), jnp.int32),
        scratch_shapes=[
            pltpu.VMEM((NPAD, D), jnp.float32),
        ],
    )(feats_pad, feats_pad)


def _make_gather():
    info = plsc.get_sparse_core_info()
    nw = info.num_cores * info.num_subcores
    bpw = NPAD // nw
    mesh = plsc.VectorSubcoreMesh(core_axis_name="c", subcore_axis_name="s")

    @functools.partial(
        pl.kernel,
        mesh=mesh,
        out_type=jax.ShapeDtypeStruct((KN, NPAD, D), jnp.float32),
        scratch_types=[
            pltpu.VMEM((bpw,), jnp.int32),
            pltpu.VMEM((bpw, D), jnp.float32),
            pltpu.SemaphoreType.DMA,
        ],
    )
    def gather_k(table_hbm, i0_hbm, i1_hbm, i2_hbm, out_hbm, idx_v, rows_v,
                 sem):
        wid = lax.axis_index("s") * info.num_cores + lax.axis_index("c")
        base = wid * bpw
        for k, idx_hbm in enumerate((i0_hbm, i1_hbm, i2_hbm)):
            pltpu.sync_copy(idx_hbm.at[pl.ds(base, bpw)], idx_v)
            pltpu.async_copy(table_hbm.at[idx_v], rows_v, sem).wait()
            pltpu.sync_copy(rows_v, out_hbm.at[k, pl.ds(base, bpw)])

    return gather_k


def _fc_body(g_ref, w_ref, p_ref, out_ref):
    g0 = g_ref[0]
    g1 = g_ref[1]
    g2 = g_ref[2]
    pooled = (g0 * p_ref[0, 0] + g1 * p_ref[0, 1] + g2 * p_ref[0, 2]
              + p_ref[0, 3])
    acc = lax.dot_general(pooled, w_ref[...], (((1,), (1,)), ((), ())),
                          preferred_element_type=jnp.float32)
    out_ref[...] = jnp.maximum(acc, 0.0)


def _run_fc(g, w_fc, params):
    return pl.pallas_call(
        _fc_body,
        grid=(NRB,),
        in_specs=[
            pl.BlockSpec((KN, RB, D), lambda r: (0, r, 0)),
            pl.BlockSpec((D, D), lambda r: (0, 0)),
            pl.BlockSpec(memory_space=pltpu.SMEM),
        ],
        out_specs=pl.BlockSpec((RB, D), lambda r: (r, 0)),
        out_shape=jax.ShapeDtypeStruct((NPAD, D), jnp.float32),
    )(g, w_fc, params)


def kernel(feats, edge_dict, W_fc, conv_w, conv_b):
    del edge_dict
    f2 = feats[0]
    fpad = jnp.pad(f2, ((0, NPAD - N), (0, 0)))

    idx8 = _run_topk(fpad)
    i0 = idx8[:, 0]
    i1 = idx8[:, 1]
    i2 = idx8[:, 2]

    g = _make_gather()(f2, i0, i1, i2)

    params = jnp.zeros((1, 8), jnp.float32)
    params = params.at[0, :KN].set(conv_w).at[0, KN].set(conv_b[0])
    out = _run_fc(g, W_fc, params)
    return out[:N][None]


# speedup vs baseline: 61.6292x; 1.0257x over previous
import functools

import jax
import jax.numpy as jnp
from jax import lax
from jax.experimental import pallas as pl
from jax.experimental.pallas import tpu as pltpu
from jax.experimental.pallas import tpu_sc as plsc

N = 10000
D = 128
KN = 3
NPAD = 10240
RB = 512
NRB = NPAD // RB
CT = 1024
NCT = NPAD // CT
NEG = float("-inf")
BIGI = 2**30


def _top3_of(v, ids, pad_id):
    vals, outs = [], []
    for t in range(3):
        m = jnp.max(v, axis=1, keepdims=True)
        am = jnp.min(jnp.where(v == m, ids, pad_id), axis=1, keepdims=True)
        vals.append(m)
        outs.append(am)
        if t < 2:
            v = jnp.where(ids == am, NEG, v)
    return vals, outs


def _topk_body(feats_full_ref, rows_ref, idx_ref, xn_ref):
    r = pl.program_id(0)

    @pl.when(r == 0)
    def _():
        for c in range(NRB):
            x = feats_full_ref[pl.ds(c * RB, RB), :]
            n2 = jnp.sum(x * x, axis=1, keepdims=True)
            xn_ref[pl.ds(c * RB, RB), :] = x / jnp.maximum(jnp.sqrt(n2), 1e-12)

    x = rows_ref[...]
    n2 = jnp.sum(x * x, axis=1, keepdims=True)
    rows = x / jnp.maximum(jnp.sqrt(n2), 1e-12)

    lid = lax.broadcasted_iota(jnp.int32, (RB, CT), 1)
    run_v, run_i = None, None

    for j in range(NCT):
        cols = xn_ref[pl.ds(j * CT, CT), :]
        s = lax.dot_general(rows, cols, (((1,), (1,)), ((), ())),
                            preferred_element_type=jnp.float32)
        if (j + 1) * CT > N:
            s = jnp.where(lid < N - j * CT, s, NEG)
        tv, ti = _top3_of(s, lid, BIGI)
        ti = [t + j * CT for t in ti]
        if j == 0:
            run_v, run_i = tv, ti
            continue
        pad = jnp.full((RB, 1), NEG, dtype=jnp.float32)
        padi = jnp.full((RB, 1), BIGI, dtype=jnp.int32)
        cv = jnp.concatenate(run_v + tv + [pad, pad], axis=1)
        ci = jnp.concatenate(run_i + ti + [padi, padi], axis=1)
        run_v, run_i = _top3_of(cv, ci, BIGI)

    padi = jnp.full((RB, 1), 0, dtype=jnp.int32)
    idx_ref[...] = jnp.concatenate(run_i + [padi] * 5, axis=1)


def _run_topk(feats_pad):
    return pl.pallas_call(
        _topk_body,
        grid=(NRB,),
        in_specs=[
            pl.BlockSpec((NPAD, D), lambda r: (0, 0)),
            pl.BlockSpec((RB, D), lambda r: (r, 0)),
        ],
        out_specs=pl.BlockSpec((RB, 8), lambda r: (r, 0)),
        out_shape=jax.ShapeDtypeStruct((NPAD, 8), jnp.int32),
        scratch_shapes=[
            pltpu.VMEM((NPAD, D), jnp.float32),
        ],
    )(feats_pad, feats_pad)


def _make_gather_pool():
    info = plsc.get_sparse_core_info()
    nw = info.num_cores * info.num_subcores
    bpw = NPAD // nw
    mesh = plsc.VectorSubcoreMesh(core_axis_name="c", subcore_axis_name="s")
    L = 16

    @functools.partial(
        pl.kernel,
        mesh=mesh,
        out_type=jax.ShapeDtypeStruct((NPAD, D), jnp.float32),
        scratch_types=[
            pltpu.VMEM((bpw,), jnp.int32),
            pltpu.VMEM((bpw,), jnp.int32),
            pltpu.VMEM((bpw,), jnp.int32),
            pltpu.VMEM((bpw, D), jnp.float32),
            pltpu.VMEM((bpw, D), jnp.float32),
            pltpu.VMEM((bpw, D), jnp.float32),
            pltpu.VMEM((4, L), jnp.float32),
            pltpu.SemaphoreType.DMA,
            pltpu.SemaphoreType.DMA,
        ],
    )
    def gather_pool(table_hbm, i0_hbm, i1_hbm, i2_hbm, cwb_hbm, out_hbm,
                    x0_v, x1_v, x2_v, r0_v, r1_v, r2_v, cw_v, isem, gsem):
        wid = lax.axis_index("s") * info.num_cores + lax.axis_index("c")
        base = wid * bpw
        pltpu.sync_copy(cwb_hbm, cw_v)
        for idx_hbm, idx_v in ((i0_hbm, x0_v), (i1_hbm, x1_v), (i2_hbm, x2_v)):
            pltpu.async_copy(idx_hbm.at[pl.ds(base, bpw)], idx_v, isem).wait()
        copies = []
        for idx_v, rows_v in ((x0_v, r0_v), (x1_v, r1_v), (x2_v, r2_v)):
            copies.append(
                pltpu.async_copy(table_hbm.at[idx_v], rows_v, gsem))
        for c in copies:
            c.wait()
        w0 = cw_v[0, :]
        w1 = cw_v[1, :]
        w2 = cw_v[2, :]
        bb = cw_v[3, :]
        def body(i, carry):
            for c in range(D // L):
                sl = pl.ds(c * L, L)
                acc = (r0_v[i, sl] * w0 + r1_v[i, sl] * w1
                       + r2_v[i, sl] * w2 + bb)
                r0_v[i, sl] = acc
            return carry
        lax.fori_loop(0, bpw, body, 0)
        pltpu.sync_copy(r0_v, out_hbm.at[pl.ds(base, bpw)])

    return gather_pool


def _fc_body(g_ref, w_ref, out_ref):
    acc = lax.dot_general(g_ref[...], w_ref[...], (((1,), (1,)), ((), ())),
                          preferred_element_type=jnp.float32)
    out_ref[...] = jnp.maximum(acc, 0.0)


def _run_fc(pooled, w_fc):
    return pl.pallas_call(
        _fc_body,
        grid=(NRB,),
        in_specs=[
            pl.BlockSpec((RB, D), lambda r: (r, 0)),
            pl.BlockSpec((D, D), lambda r: (0, 0)),
        ],
        out_specs=pl.BlockSpec((RB, D), lambda r: (r, 0)),
        out_shape=jax.ShapeDtypeStruct((NPAD, D), jnp.float32),
    )(pooled, w_fc)


def kernel(feats, edge_dict, W_fc, conv_w, conv_b):
    del edge_dict
    f2 = feats[0]
    fpad = jnp.pad(f2, ((0, NPAD - N), (0, 0)))

    idx8 = _run_topk(fpad)
    i0 = idx8[:, 0]
    i1 = idx8[:, 1]
    i2 = idx8[:, 2]

    cwb = jnp.broadcast_to(
        jnp.concatenate([conv_w, conv_b])[:, None], (KN + 1, 16))
    pooled = _make_gather_pool()(f2, i0, i1, i2, cwb)

    out = _run_fc(pooled, W_fc)
    return out[:N][None]
